# baseline (speedup 1.0000x reference)
import functools

import jax
import jax.numpy as jnp
from jax import lax
from jax.experimental import pallas as pl
from jax.experimental.pallas import tpu as pltpu
from jax.experimental.pallas import tpu_sc as plsc

VOCAB1 = 100001
EMB = 32
BATCH = 1024
HIST = 50
HIST_PAD = 52

NC = 2
NS = 16
NW = NC * NS
ROWS_PER_W = BATCH // NW
IDX_PER_W = ROWS_PER_W * HIST_PAD
IDX_COLS = 128
IDX_ROWS_W = IDX_PER_W // IDX_COLS


def _sc_sums(ctx2p, table):
    mesh = plsc.VectorSubcoreMesh(
        core_axis_name="c", subcore_axis_name="s", num_cores=NC, num_subcores=NS
    )

    @functools.partial(
        pl.kernel,
        out_type=(
            jax.ShapeDtypeStruct((BATCH, EMB), jnp.float32),
            jax.ShapeDtypeStruct((BATCH,), jnp.float32),
        ),
        mesh=mesh,
        compiler_params=pltpu.CompilerParams(
            needs_layout_passes=False, use_tc_tiling_on_sc=False
        ),
        scratch_types=[
            pltpu.VMEM((IDX_ROWS_W, IDX_COLS), jnp.int32),
            pltpu.VMEM((1, IDX_COLS), jnp.int32),
            pltpu.VMEM((IDX_PER_W, EMB), jnp.float32),
            pltpu.VMEM_SHARED((NS * ROWS_PER_W, EMB), jnp.float32),
            pltpu.VMEM((ROWS_PER_W, EMB), jnp.float32),
            pltpu.VMEM((ROWS_PER_W,), jnp.float32),
            pltpu.SemaphoreType.DMA,
            pltpu.SemaphoreType.DMA,
        ],
    )
    def k(ctx_hbm, table_hbm, qraw_hbm, cnt_hbm,
          idx_v, seg_v, rows_v, qacc_sh, qtmp_v, cnt_v, sem_g, sem_s):
        sid = lax.axis_index("s")
        wid = sid * NC + lax.axis_index("c")
        arow = sid * ROWS_PER_W

        pltpu.sync_copy(ctx_hbm.at[pl.ds(wid * IDX_ROWS_W, IDX_ROWS_W)], idx_v)

        gathers = [
            pltpu.async_copy(
                table_hbm.at[idx_v.at[j]],
                rows_v.at[pl.ds(j * IDX_COLS, IDX_COLS)],
                sem_g,
            )
            for j in range(IDX_ROWS_W)
        ]

        lanes = lax.iota(jnp.int32, 16)
        for t in range(IDX_COLS // 16):
            seg_v[0, pl.ds(t * 16, 16)] = lanes + (arow + (16 if t % 2 else 0))

        zero = jnp.zeros((16,), jnp.float32)
        for i in range(ROWS_PER_W):
            qtmp_v[i, pl.ds(0, 16)] = zero
            qtmp_v[i, pl.ds(16, 16)] = zero
        pltpu.sync_copy(qtmp_v, qacc_sh.at[pl.ds(arow, ROWS_PER_W)])

        cnt_a = jnp.zeros((16,), jnp.float32)
        cnt_b = jnp.zeros((16,), jnp.float32)
        one = jnp.full((16,), 1.0, jnp.float32)
        for j in range(IDX_ROWS_W):
            for t in range(IDX_COLS // 16):
                ids = idx_v[j, pl.ds(t * 16, 16)]
                m = jnp.where(ids != 0, one, zero)
                if t % 2 == 0:
                    cnt_a = cnt_a + m
                else:
                    cnt_b = cnt_b + m
        cnt_v[pl.ds(0, 16)] = cnt_a
        cnt_v[pl.ds(16, 16)] = cnt_b

        for g in gathers:
            g.wait()
        scatters = [
            pltpu.async_copy(
                rows_v.at[pl.ds(j * IDX_COLS, IDX_COLS)],
                qacc_sh.at[seg_v.at[0]],
                sem_s,
                add=True,
            )
            for j in range(IDX_ROWS_W)
        ]
        for s in scatters:
            s.wait()

        pltpu.sync_copy(
            qacc_sh.at[pl.ds(arow, ROWS_PER_W)],
            qraw_hbm.at[pl.ds(wid * ROWS_PER_W, ROWS_PER_W)],
        )
        pltpu.sync_copy(cnt_v, cnt_hbm.at[pl.ds(wid * ROWS_PER_W, ROWS_PER_W)])

    return k(ctx2p, table)


BT = 2048
NTB = pl.cdiv(VOCAB1, BT)


def _tc_label_t(label_table):
    def tk(l_ref, o_ref):
        o_ref[...] = l_ref[...].T.astype(jnp.bfloat16)

    return pl.pallas_call(
        tk,
        grid=(NTB,),
        in_specs=[pl.BlockSpec((BT, EMB), lambda n: (n, 0))],
        out_specs=pl.BlockSpec((EMB, BT), lambda n: (0, n)),
        out_shape=jax.ShapeDtypeStruct((EMB, VOCAB1), jnp.bfloat16),
    )(label_table)


BM = 32
NMB = BATCH // BM
NSLOT = 3


def _tc_scores(qraw, cnt, g, label16t):
    def mm(q_ref, c_ref, g_ref, l_ref, o_hbm, obuf, sems):
        m = pl.program_id(0)
        slot = lax.rem(m, NSLOT)

        cntc = c_ref[...]
        recip = 1.0 / jnp.maximum(cntc, 1.0)
        npad = jnp.float32(HIST_PAD) - cntc
        q = (q_ref[...] - npad * g_ref[...]) * recip
        res = lax.dot_general(
            q.astype(jnp.bfloat16), l_ref[...],
            (((1,), (0,)), ((), ())),
            preferred_element_type=jnp.float32,
        )

        def desc(s):
            return pltpu.make_async_copy(
                obuf.at[s],
                o_hbm.at[pl.ds(m * BM, BM)],
                sems.at[s],
            )

        for s in range(NSLOT):
            @pl.when((slot == s) & (m >= NSLOT))
            def _():
                desc(s).wait()

        for s in range(NSLOT):
            @pl.when(slot == s)
            def _():
                obuf[s, :, :] = res

        for s in range(NSLOT):
            @pl.when(slot == s)
            def _():
                desc(s).start()

        @pl.when(m == NMB - 1)
        def _():
            for s in range(NSLOT):
                desc(s).wait()

    return pl.pallas_call(
        mm,
        grid=(NMB,),
        in_specs=[
            pl.BlockSpec((BM, EMB), lambda m: (m, 0)),
            pl.BlockSpec((BM, 1), lambda m: (m, 0)),
            pl.BlockSpec((1, EMB), lambda m: (0, 0)),
            pl.BlockSpec((EMB, VOCAB1), lambda m: (0, 0)),
        ],
        out_specs=pl.BlockSpec(memory_space=pl.ANY),
        out_shape=jax.ShapeDtypeStruct((BATCH, VOCAB1), jnp.float32),
        scratch_shapes=[
            pltpu.VMEM((NSLOT, BM, VOCAB1), jnp.float32),
            pltpu.SemaphoreType.DMA((NSLOT,)),
        ],
    )(qraw, cnt, g, label16t)


def kernel(context, context_table, label_table):
    ctx_pad = jnp.pad(context, ((0, 0), (0, HIST_PAD - HIST)))
    ctx2p = (
        ctx_pad.reshape(NW, ROWS_PER_W, HIST_PAD)
        .transpose(0, 2, 1)
        .reshape(NW * IDX_ROWS_W, IDX_COLS)
    )
    qraw, cnt = _sc_sums(ctx2p, context_table)
    label16t = _tc_label_t(label_table)
    return _tc_scores(qraw, cnt.reshape(BATCH, 1), context_table[0:1], label16t)


# speedup vs baseline: 2.8969x; 2.8969x over previous
import functools

import jax
import jax.numpy as jnp
from jax import lax
from jax.experimental import pallas as pl
from jax.experimental.pallas import tpu as pltpu
from jax.experimental.pallas import tpu_sc as plsc

VOCAB1 = 100001
EMB = 32
BATCH = 1024
HIST = 50
HIST_PAD = 52

NC = 2
NS = 16
NW = NC * NS
ROWS_PER_W = BATCH // NW
IDX_PER_W = ROWS_PER_W * HIST_PAD
IDX_COLS = 128
IDX_ROWS_W = IDX_PER_W // IDX_COLS


def _sc_sums(ctx2p, table):
    mesh = plsc.VectorSubcoreMesh(
        core_axis_name="c", subcore_axis_name="s", num_cores=NC, num_subcores=NS
    )

    @functools.partial(
        pl.kernel,
        out_type=(
            jax.ShapeDtypeStruct((BATCH, EMB), jnp.float32),
            jax.ShapeDtypeStruct((BATCH,), jnp.float32),
        ),
        mesh=mesh,
        compiler_params=pltpu.CompilerParams(
            needs_layout_passes=False, use_tc_tiling_on_sc=False
        ),
        scratch_types=[
            pltpu.VMEM((IDX_ROWS_W, IDX_COLS), jnp.int32),
            pltpu.VMEM((1, IDX_COLS), jnp.int32),
            pltpu.VMEM((IDX_PER_W, EMB), jnp.float32),
            pltpu.VMEM_SHARED((NS * ROWS_PER_W, EMB), jnp.float32),
            pltpu.VMEM((ROWS_PER_W, EMB), jnp.float32),
            pltpu.VMEM((ROWS_PER_W,), jnp.float32),
            pltpu.SemaphoreType.DMA,
            pltpu.SemaphoreType.DMA,
        ],
    )
    def k(ctx_hbm, table_hbm, qraw_hbm, cnt_hbm,
          idx_v, seg_v, rows_v, qacc_sh, qtmp_v, cnt_v, sem_g, sem_s):
        sid = lax.axis_index("s")
        wid = sid * NC + lax.axis_index("c")
        arow = sid * ROWS_PER_W

        pltpu.sync_copy(ctx_hbm.at[pl.ds(wid * IDX_ROWS_W, IDX_ROWS_W)], idx_v)

        gathers = [
            pltpu.async_copy(
                table_hbm.at[idx_v.at[j]],
                rows_v.at[pl.ds(j * IDX_COLS, IDX_COLS)],
                sem_g,
            )
            for j in range(IDX_ROWS_W)
        ]

        lanes = lax.iota(jnp.int32, 16)
        for t in range(IDX_COLS // 16):
            seg_v[0, pl.ds(t * 16, 16)] = lanes + (arow + (16 if t % 2 else 0))

        zero = jnp.zeros((16,), jnp.float32)
        for i in range(ROWS_PER_W):
            qtmp_v[i, pl.ds(0, 16)] = zero
            qtmp_v[i, pl.ds(16, 16)] = zero
        pltpu.sync_copy(qtmp_v, qacc_sh.at[pl.ds(arow, ROWS_PER_W)])

        cnt_a = jnp.zeros((16,), jnp.float32)
        cnt_b = jnp.zeros((16,), jnp.float32)
        one = jnp.full((16,), 1.0, jnp.float32)
        for j in range(IDX_ROWS_W):
            for t in range(IDX_COLS // 16):
                ids = idx_v[j, pl.ds(t * 16, 16)]
                m = jnp.where(ids != 0, one, zero)
                if t % 2 == 0:
                    cnt_a = cnt_a + m
                else:
                    cnt_b = cnt_b + m
        cnt_v[pl.ds(0, 16)] = cnt_a
        cnt_v[pl.ds(16, 16)] = cnt_b

        for g in gathers:
            g.wait()
        scatters = [
            pltpu.async_copy(
                rows_v.at[pl.ds(j * IDX_COLS, IDX_COLS)],
                qacc_sh.at[seg_v.at[0]],
                sem_s,
                add=True,
            )
            for j in range(IDX_ROWS_W)
        ]
        for s in scatters:
            s.wait()

        pltpu.sync_copy(
            qacc_sh.at[pl.ds(arow, ROWS_PER_W)],
            qraw_hbm.at[pl.ds(wid * ROWS_PER_W, ROWS_PER_W)],
        )
        pltpu.sync_copy(cnt_v, cnt_hbm.at[pl.ds(wid * ROWS_PER_W, ROWS_PER_W)])

    return k(ctx2p, table)


BV = 2048
NVB = pl.cdiv(VOCAB1, BV)


def _tc_scores_t(qraw, cnt, g, label_table):
    def mm(lt_ref, q_ref, c_ref, g_ref, o_ref):
        cntc = c_ref[...]
        recip = 1.0 / jnp.maximum(cntc, 1.0)
        npad = jnp.float32(HIST_PAD) - cntc
        q = (q_ref[...] - npad * g_ref[...]) * recip
        o_ref[...] = lax.dot_general(
            lt_ref[...].astype(jnp.bfloat16), q.astype(jnp.bfloat16),
            (((0,), (1,)), ((), ())),
            preferred_element_type=jnp.float32,
        )

    return pl.pallas_call(
        mm,
        grid=(NVB,),
        in_specs=[
            pl.BlockSpec((EMB, BV), lambda n: (0, n)),
            pl.BlockSpec((BATCH, EMB), lambda n: (0, 0)),
            pl.BlockSpec((BATCH, 1), lambda n: (0, 0)),
            pl.BlockSpec((1, EMB), lambda n: (0, 0)),
        ],
        out_specs=pl.BlockSpec((BV, BATCH), lambda n: (n, 0)),
        out_shape=jax.ShapeDtypeStruct((VOCAB1, BATCH), jnp.float32),
    )(label_table.T, qraw, cnt, g)


def kernel(context, context_table, label_table):
    ctx_pad = jnp.pad(context, ((0, 0), (0, HIST_PAD - HIST)))
    ctx2p = (
        ctx_pad.reshape(NW, ROWS_PER_W, HIST_PAD)
        .transpose(0, 2, 1)
        .reshape(NW * IDX_ROWS_W, IDX_COLS)
    )
    qraw, cnt = _sc_sums(ctx2p, context_table)
    out_t = _tc_scores_t(qraw, cnt.reshape(BATCH, 1), context_table[0:1], label_table)
    return out_t.T
